# Initial kernel scaffold; baseline (speedup 1.0000x reference)
import jax
import jax.numpy as jnp
from jax.experimental import pallas as pl


def kernel(xyz, points, affine_alpha, affine_beta):
    raise NotImplementedError("write your pallas kernel here")


# speedup vs baseline: 1.6447x; 1.6447x over previous
import functools

import jax
import jax.numpy as jnp
from jax.experimental import pallas as pl
from jax.experimental.pallas import tpu as pltpu

_B, _N, _D = 8, 4096, 128
_S, _K = 1024, 32


def _fps_body(xyzT_ref, out_ref):
    x = xyzT_ref[0]
    y = xyzT_ref[1]
    z = xyzT_ref[2]
    lane = jax.lax.broadcasted_iota(jnp.int32, (_B, _N), 1)
    lane_s = jax.lax.broadcasted_iota(jnp.int32, (_B, _S), 1)
    out_ref[...] = jnp.zeros((_B, _S), jnp.int32)

    def step(i, carry):
        dist, far = carry
        out_ref[...] = out_ref[...] + jnp.where(lane_s == i, 1, 0) * far
        sel = lane == far
        cx = jnp.sum(jnp.where(sel, x, 0.0), axis=1, keepdims=True)
        cy = jnp.sum(jnp.where(sel, y, 0.0), axis=1, keepdims=True)
        cz = jnp.sum(jnp.where(sel, z, 0.0), axis=1, keepdims=True)
        dx = x - cx
        dy = y - cy
        dz = z - cz
        d = (dx * dx + dy * dy) + dz * dz
        dist = jnp.minimum(dist, d)
        m = jnp.max(dist, axis=1, keepdims=True)
        far = jnp.min(jnp.where(dist == m, lane, _N), axis=1, keepdims=True)
        return dist, far.astype(jnp.int32)

    init = (
        jnp.full((_B, _N), 1e10, jnp.float32),
        jnp.zeros((_B, 1), jnp.int32),
    )
    jax.lax.fori_loop(0, _S, step, init)


def _fps(xyz):
    xyzT = jnp.transpose(xyz, (2, 0, 1))
    return pl.pallas_call(
        _fps_body,
        out_shape=jax.ShapeDtypeStruct((_B, _S), jnp.int32),
    )(xyzT)


def _index_points(points, idx):
    return jax.vmap(lambda p, i: p[i])(points, idx)


def kernel(xyz, points, affine_alpha, affine_beta):
    b = xyz.shape[0]
    fps_idx = _fps(xyz)
    new_xyz = _index_points(xyz, fps_idx)
    new_points = _index_points(points, fps_idx)

    dist = -2.0 * jnp.matmul(new_xyz, jnp.swapaxes(xyz, 1, 2))
    dist = dist + jnp.sum(new_xyz ** 2, -1)[:, :, None]
    dist = dist + jnp.sum(xyz ** 2, -1)[:, None, :]
    _, idx = jax.lax.top_k(-dist, _K)

    grouped_xyz = _index_points(xyz, idx)
    grouped_points = _index_points(points, idx)
    grouped_points = jnp.concatenate([grouped_points, grouped_xyz], axis=-1)
    mean = jnp.concatenate([new_points, new_xyz], axis=-1)[:, :, None, :]
    std = jnp.std((grouped_points - mean).reshape(b, -1), axis=-1, ddof=1)[
        :, None, None, None
    ]
    grouped_points = (grouped_points - mean) / (std + 1e-05)
    grouped_points = affine_alpha * grouped_points + affine_beta
    rep = jnp.broadcast_to(
        new_points[:, :, None, :], (b, _S, _K, points.shape[-1])
    )
    new_points_out = jnp.concatenate([grouped_points, rep], axis=-1)
    return (new_xyz, new_points_out)
